# baseline (speedup 1.0000x reference)
import jax
import jax.numpy as jnp
from jax import lax
from jax.experimental import pallas as pl
from jax.experimental.pallas import tpu as pltpu
from jax.experimental.pallas import tpu_sc as plsc

_B = 16384
_D = 64
_L = 16
_NC = 2
_NS = 16
_NW = _NC * _NS
_BPW = _B // _NW
_CH = 128
_NCH = _BPW // _CH
_NPAIR = 500000


def _bpr_body(upair_hbm, ipair_hbm, uhalf_hbm, ihalf_hbm, uemb_hbm, iemb_hbm,
              out_hbm, upair_v, ipair_v, uhalf_v, ihalf_v, urows_v, irows_v,
              out_v, sem):
    wid = lax.axis_index("s") * _NC + lax.axis_index("c")
    base = wid * _BPW

    pltpu.sync_copy(upair_hbm.at[wid], upair_v)
    pltpu.sync_copy(ipair_hbm.at[wid], ipair_v)
    pltpu.sync_copy(uhalf_hbm.at[wid], uhalf_v)
    pltpu.sync_copy(ihalf_hbm.at[wid], ihalf_v)

    lane = lax.iota(jnp.int32, _L)
    gat_dnums = lax.GatherDimensionNumbers(
        offset_dims=(), collapsed_slice_dims=(0,), start_index_map=(0,))
    rot_idx = [jnp.bitwise_and(lane + sh, _L - 1) for sh in (8, 4, 2, 1)]

    def _lane_rotate(p, idx):
        return lax.gather(p, idx[:, None], gat_dnums, (1,),
                          mode=lax.GatherScatterMode.PROMISE_IN_BOUNDS)

    def chunk(j, carry):
        cu = pltpu.async_copy(uemb_hbm.at[upair_v.at[j]], urows_v, sem)
        ci = pltpu.async_copy(iemb_hbm.at[ipair_v.at[j]], irows_v, sem)
        cu.wait()
        ci.wait()
        for g in range(_CH // _L):
            suv = uhalf_v[j, pl.ds(g * _L, _L)]
            siv = ihalf_v[j, pl.ds(g * _L, _L)]
            dots = jnp.zeros((_L,), jnp.float32)
            for k in range(_L):
                kk = g * _L + k
                uo = suv[k] * _D
                io = siv[k] * _D
                p = (urows_v[kk, pl.ds(uo, _L)]
                     * irows_v[kk, pl.ds(io, _L)])
                for c in range(1, _D // _L):
                    p = p + (urows_v[kk, pl.ds(uo + c * _L, _L)]
                             * irows_v[kk, pl.ds(io + c * _L, _L)])
                for idx in rot_idx:
                    p = p + _lane_rotate(p, idx)
                dots = jnp.where(lane == k, p, dots)
            out_v[pl.ds(j * _CH + g * _L, _L)] = dots
        return carry

    lax.fori_loop(0, _NCH, chunk, 0)
    pltpu.sync_copy(out_v, out_hbm.at[pl.ds(base, _BPW)])


def kernel(users, items, user_emb, item_emb):
    users = users.astype(jnp.int32)
    items = items.astype(jnp.int32)
    upair = (users >> 1).reshape(_NW, _NCH, _CH)
    ipair = (items >> 1).reshape(_NW, _NCH, _CH)
    uhalf = (users & 1).reshape(_NW, _NCH, _CH)
    ihalf = (items & 1).reshape(_NW, _NCH, _CH)
    uemb2 = user_emb.reshape(_NPAIR, 2 * _D)
    iemb2 = item_emb.reshape(_NPAIR, 2 * _D)
    mesh = plsc.VectorSubcoreMesh(core_axis_name="c", subcore_axis_name="s")
    run = pl.kernel(
        _bpr_body,
        out_type=jax.ShapeDtypeStruct((_B,), jnp.float32),
        mesh=mesh,
        compiler_params=pltpu.CompilerParams(use_tc_tiling_on_sc=False),
        scratch_types=[
            pltpu.VMEM((_NCH, _CH), jnp.int32),
            pltpu.VMEM((_NCH, _CH), jnp.int32),
            pltpu.VMEM((_NCH, _CH), jnp.int32),
            pltpu.VMEM((_NCH, _CH), jnp.int32),
            pltpu.VMEM((_CH, 2 * _D), jnp.float32),
            pltpu.VMEM((_CH, 2 * _D), jnp.float32),
            pltpu.VMEM((_BPW,), jnp.float32),
            pltpu.SemaphoreType.DMA,
        ],
    )
    return run(upair, ipair, uhalf, ihalf, uemb2, iemb2)


# speedup vs baseline: 1.0002x; 1.0002x over previous
import jax
import jax.numpy as jnp
from jax import lax
from jax.experimental import pallas as pl
from jax.experimental.pallas import tpu as pltpu
from jax.experimental.pallas import tpu_sc as plsc

_B = 16384
_D = 64
_L = 16
_NC = 2
_NS = 16
_NW = _NC * _NS
_BPW = _B // _NW
_CH = 128
_NCH = _BPW // _CH
_NPAIR = 500000


def _bpr_body(upair_hbm, ipair_hbm, uhalf_hbm, ihalf_hbm, uemb_hbm, iemb_hbm,
              out_hbm, upair_v, ipair_v, uhalf_v, ihalf_v, urows_v, irows_v,
              out_v, sem):
    wid = lax.axis_index("s") * _NC + lax.axis_index("c")
    base = wid * _BPW

    pltpu.sync_copy(upair_hbm.at[wid], upair_v)
    pltpu.sync_copy(ipair_hbm.at[wid], ipair_v)
    pltpu.sync_copy(uhalf_hbm.at[wid], uhalf_v)
    pltpu.sync_copy(ihalf_hbm.at[wid], ihalf_v)

    lane = lax.iota(jnp.int32, _L)
    gat_dnums = lax.GatherDimensionNumbers(
        offset_dims=(), collapsed_slice_dims=(0,), start_index_map=(0,))
    rot_idx = [jnp.bitwise_and(lane + sh, _L - 1) for sh in (8, 4, 2, 1)]

    def _lane_rotate(p, idx):
        return lax.gather(p, idx[:, None], gat_dnums, (1,),
                          mode=lax.GatherScatterMode.PROMISE_IN_BOUNDS)

    def chunk(j, carry):
        cu = pltpu.async_copy(uemb_hbm.at[upair_v.at[j]], urows_v, sem)
        ci = pltpu.async_copy(iemb_hbm.at[ipair_v.at[j]], irows_v, sem)
        cu.wait()
        ci.wait()
        for g in range(_CH // _L):
            suv = uhalf_v[j, pl.ds(g * _L, _L)]
            siv = ihalf_v[j, pl.ds(g * _L, _L)]
            dots = jnp.zeros((_L,), jnp.float32)
            for k in range(_L):
                kk = g * _L + k
                uo = suv[k] * _D
                io = siv[k] * _D
                p = (urows_v[kk, pl.ds(uo, _L)]
                     * irows_v[kk, pl.ds(io, _L)])
                for c in range(1, _D // _L):
                    p = p + (urows_v[kk, pl.ds(uo + c * _L, _L)]
                             * irows_v[kk, pl.ds(io + c * _L, _L)])
                for idx in rot_idx:
                    p = p + _lane_rotate(p, idx)
                dots = jnp.where(lane == k, p, dots)
            out_v[pl.ds(j * _CH + g * _L, _L)] = dots
        return carry

    lax.fori_loop(0, _NCH, chunk, 0)
    pltpu.sync_copy(out_v, out_hbm.at[pl.ds(base, _BPW)])


def kernel(users, items, user_emb, item_emb):
    users = users.astype(jnp.int32)
    items = items.astype(jnp.int32)
    upair = (users >> 1).reshape(_NW, _NCH, _CH)
    ipair = (items >> 1).reshape(_NW, _NCH, _CH)
    uhalf = (users & 1).reshape(_NW, _NCH, _CH)
    ihalf = (items & 1).reshape(_NW, _NCH, _CH)
    uemb2 = user_emb.reshape(_NPAIR, 2 * _D)
    iemb2 = item_emb.reshape(_NPAIR, 2 * _D)
    mesh = plsc.VectorSubcoreMesh(core_axis_name="c", subcore_axis_name="s")
    run = pl.kernel(
        _bpr_body,
        out_type=jax.ShapeDtypeStruct((_B,), jnp.float32),
        mesh=mesh,
        scratch_types=[
            pltpu.VMEM((_NCH, _CH), jnp.int32),
            pltpu.VMEM((_NCH, _CH), jnp.int32),
            pltpu.VMEM((_NCH, _CH), jnp.int32),
            pltpu.VMEM((_NCH, _CH), jnp.int32),
            pltpu.VMEM((_CH, 2 * _D), jnp.float32),
            pltpu.VMEM((_CH, 2 * _D), jnp.float32),
            pltpu.VMEM((_BPW,), jnp.float32),
            pltpu.SemaphoreType.DMA,
        ],
    )
    return run(upair, ipair, uhalf, ihalf, uemb2, iemb2)


# speedup vs baseline: 1.4519x; 1.4516x over previous
import jax
import jax.numpy as jnp
from jax import lax
from jax.experimental import pallas as pl
from jax.experimental.pallas import tpu as pltpu
from jax.experimental.pallas import tpu_sc as plsc

_B = 16384
_D = 64
_L = 16
_NC = 2
_NS = 16
_NW = _NC * _NS
_BPW = _B // _NW
_CH = 128
_NCH = _BPW // _CH
_NBLK = 125000


def _bpr_body(uidx_hbm, itid_hbm, isub_hbm, uemb_hbm, iemb_hbm,
              out_hbm, uidx_v, itid_v, isub_v, urows_v, iblk_v,
              out_v, usem, isem):
    wid = lax.axis_index("s") * _NC + lax.axis_index("c")
    base = wid * _BPW

    pltpu.sync_copy(uidx_hbm.at[wid], uidx_v)
    pltpu.sync_copy(itid_hbm.at[wid], itid_v)
    pltpu.sync_copy(isub_hbm.at[wid], isub_v)

    lane = lax.iota(jnp.int32, _L)
    gat_dnums = lax.GatherDimensionNumbers(
        offset_dims=(), collapsed_slice_dims=(0,), start_index_map=(0,))
    rot_idx = [jnp.bitwise_and(lane + sh, _L - 1) for sh in (8, 4, 2, 1)]

    def _lane_rotate(p, idx):
        return lax.gather(p, idx[:, None], gat_dnums, (1,),
                          mode=lax.GatherScatterMode.PROMISE_IN_BOUNDS)

    def chunk(j, carry):
        cu = pltpu.async_copy(uemb_hbm.at[uidx_v.at[j]], urows_v, usem)
        cu.wait()
        for g in range(_CH // _L):
            itv = itid_v[pl.ds(j * _CH + g * _L, _L)]
            copies = []
            for k in range(_L):
                copies.append(pltpu.async_copy(
                    iemb_hbm.at[itv[k]], iblk_v.at[k], isem))
            for c in copies:
                c.wait()

            siv = isub_v[pl.ds(j * _CH + g * _L, _L)]
            dots = jnp.zeros((_L,), jnp.float32)
            for k in range(_L):
                kk = g * _L + k
                si = siv[k]
                p = (urows_v[kk, pl.ds(0, _L)]
                     * iblk_v[k, si, pl.ds(0, _L)])
                for c in range(1, _D // _L):
                    p = p + (urows_v[kk, pl.ds(c * _L, _L)]
                             * iblk_v[k, si, pl.ds(c * _L, _L)])
                for idx in rot_idx:
                    p = p + _lane_rotate(p, idx)
                dots = jnp.where(lane == k, p, dots)
            out_v[pl.ds(j * _CH + g * _L, _L)] = dots
        return carry

    lax.fori_loop(0, _NCH, chunk, 0)
    pltpu.sync_copy(out_v, out_hbm.at[pl.ds(base, _BPW)])


def kernel(users, items, user_emb, item_emb):
    users = users.astype(jnp.int32)
    items = items.astype(jnp.int32)
    uidx = users.reshape(_NW, _NCH, _CH)
    itid = (items >> 3).reshape(_NW, _BPW)
    isub = (items & 7).reshape(_NW, _BPW)
    uemb128 = jnp.pad(user_emb, ((0, 0), (0, 128 - _D)))
    iemb3 = item_emb.reshape(_NBLK, 8, _D)
    mesh = plsc.VectorSubcoreMesh(core_axis_name="c", subcore_axis_name="s")
    run = pl.kernel(
        _bpr_body,
        out_type=jax.ShapeDtypeStruct((_B,), jnp.float32),
        mesh=mesh,
        scratch_types=[
            pltpu.VMEM((_NCH, _CH), jnp.int32),
            pltpu.VMEM((_BPW,), jnp.int32),
            pltpu.VMEM((_BPW,), jnp.int32),
            pltpu.VMEM((_CH, 128), jnp.float32),
            pltpu.VMEM((_L, 8, _D), jnp.float32),
            pltpu.VMEM((_BPW,), jnp.float32),
            pltpu.SemaphoreType.DMA,
            pltpu.SemaphoreType.DMA,
        ],
    )
    return run(uidx, itid, isub, uemb128, iemb3)
